# baseline (speedup 1.0000x reference)
import jax
import jax.numpy as jnp
from jax import lax
from jax.experimental import pallas as pl
from jax.experimental.pallas import tpu as pltpu
from jax.experimental.pallas import tpu_sc as plsc
import functools

NC, NS, L = 2, 16, 16
NW = NC * NS
B = 16384
F = 26
OFFS = 100000
BPW = B // NW
E = BPW * F
IDX_MINOR = 128
IDX_ROWS = E // IDX_MINOR
RPF = BPW // IDX_MINOR
CPW = BPW // L


@functools.partial(
    pl.kernel,
    out_type=jax.ShapeDtypeStruct((B,), jnp.float32),
    mesh=plsc.VectorSubcoreMesh(core_axis_name="c", subcore_axis_name="s"),
    scratch_types=[
        pltpu.VMEM((E,), jnp.int32),
        pltpu.VMEM((E,), jnp.float32),
        pltpu.VMEM((BPW,), jnp.float32),
        pltpu.VMEM((L,), jnp.float32),
        pltpu.SemaphoreType.DMA,
        pltpu.SemaphoreType.DMA,
    ],
    compiler_params=pltpu.CompilerParams(
        skip_device_barrier=True,
        disable_bounds_checks=True,
        disable_semaphore_checks=True,
    ),
)
def _fl_kernel(xt_hbm, table_hbm, bias_hbm, out_hbm, idxb, gb, outb, biasb, xsem, gsem):
    wid = lax.axis_index("s") * NC + lax.axis_index("c")
    base_b = wid * BPW

    pltpu.sync_copy(bias_hbm, biasb)

    @pl.loop(0, F, unroll=2)
    def _xfire(f):
        pltpu.make_async_copy(
            xt_hbm.at[f, pl.ds(base_b, BPW)],
            idxb.at[pl.ds(f * BPW, BPW)], xsem).start()

    @pl.loop(0, F, unroll=2)
    def _xdrain(f):
        pltpu.make_async_copy(
            xt_hbm.at[f, pl.ds(base_b, BPW)],
            idxb.at[pl.ds(f * BPW, BPW)], xsem).wait()

    tbl = table_hbm.at[0]

    @pl.loop(0, F)
    def _off(f):
        off = f * OFFS

        @pl.loop(0, CPW, unroll=4)
        def _offc(c):
            p = f * BPW + c * L
            idxb[pl.ds(p, L)] = idxb[pl.ds(p, L)] + off

    pltpu.async_copy(tbl.at[idxb], gb, gsem).wait()

    @pl.loop(0, CPW)
    def _reduce(c):
        acc = biasb[...]
        for f in range(F):
            p = f * BPW + c * L
            acc = acc + gb[pl.ds(p, L)]
        outb[pl.ds(c * L, L)] = acc

    pltpu.sync_copy(outb, out_hbm.at[pl.ds(base_b, BPW)])


def kernel(x, table, bias):
    b16 = jnp.full((L,), bias[0], dtype=jnp.float32)
    out = _fl_kernel(x.T, table.T, b16)
    return out.reshape(B, 1)
